# baseline (speedup 1.0000x reference)
import functools

import jax
import jax.numpy as jnp
from jax import lax
from jax.experimental import pallas as pl
from jax.experimental.pallas import tpu as pltpu
from jax.experimental.pallas import tpu_sc as plsc

B = 16384
N_FIELDS = 26
NC = 2
NS = 16
NW = NC * NS
BPW = B // NW
L = 16


@functools.partial(
    pl.kernel,
    out_type=jax.ShapeDtypeStruct((B,), jnp.float32),
    mesh=plsc.VectorSubcoreMesh(core_axis_name="c", subcore_axis_name="s"),
    scratch_types=[
        pltpu.VMEM((N_FIELDS * BPW,), jnp.int32),
        pltpu.VMEM((N_FIELDS * BPW,), jnp.float32),
        pltpu.VMEM((BPW,), jnp.float32),
        pltpu.SemaphoreType.DMA,
    ],
)
def _lookup_sum(xt_hbm, tbl_hbm, out_hbm, idx_v, vals_v, acc_v, sem):
    wid = lax.axis_index("s") * NC + lax.axis_index("c")
    base = wid * BPW
    idx_copies = [
        pltpu.async_copy(
            xt_hbm.at[j, pl.ds(base, BPW)], idx_v.at[pl.ds(j * BPW, BPW)], sem
        )
        for j in range(N_FIELDS)
    ]
    copies = []
    for j in range(0, N_FIELDS, 2):
        idx_copies[j].wait()
        idx_copies[j + 1].wait()
        copies.append(
            pltpu.async_copy(
                tbl_hbm.at[0].at[idx_v.at[pl.ds(j * BPW, 2 * BPW)]],
                vals_v.at[pl.ds(j * BPW, 2 * BPW)],
                sem,
            )
        )
    for c in copies:
        c.wait()
    for c in range(BPW // L):
        s = vals_v[pl.ds(c * L, L)]
        for j in range(1, N_FIELDS):
            s = s + vals_v[pl.ds(j * BPW + c * L, L)]
        acc_v[pl.ds(c * L, L)] = s
    pltpu.sync_copy(acc_v, out_hbm.at[pl.ds(base, BPW)])


def kernel(X, table):
    xt = X.T.reshape(N_FIELDS, B)
    out = _lookup_sum(xt, table.reshape(1, -1))
    return out.reshape(B, 1)


# speedup vs baseline: 1.0220x; 1.0220x over previous
import functools

import jax
import jax.numpy as jnp
from jax import lax
from jax.experimental import pallas as pl
from jax.experimental.pallas import tpu as pltpu
from jax.experimental.pallas import tpu_sc as plsc

B = 16384
N_FIELDS = 26
NC = 2
NS = 16
NW = NC * NS
BPW = B // NW
L = 16


@functools.partial(
    pl.kernel,
    out_type=jax.ShapeDtypeStruct((B,), jnp.float32),
    mesh=plsc.VectorSubcoreMesh(core_axis_name="c", subcore_axis_name="s"),
    scratch_types=[
        pltpu.VMEM((N_FIELDS * BPW,), jnp.int32),
        pltpu.VMEM((N_FIELDS * BPW,), jnp.float32),
        pltpu.VMEM((BPW,), jnp.float32),
        pltpu.SemaphoreType.DMA,
    ],
)
def _lookup_sum(xt_hbm, tbl_hbm, out_hbm, idx_v, vals_v, acc_v, sem):
    wid = lax.axis_index("s") * NC + lax.axis_index("c")
    base = wid * BPW
    idx_copies = [
        pltpu.async_copy(
            xt_hbm.at[j, pl.ds(base, BPW)], idx_v.at[pl.ds(j * BPW, BPW)], sem
        )
        for j in range(N_FIELDS)
    ]
    for c in idx_copies:
        c.wait()
    copies = []
    for j in range(0, N_FIELDS, 2):
        copies.append(
            pltpu.async_copy(
                tbl_hbm.at[0].at[idx_v.at[pl.ds(j * BPW, 2 * BPW)]],
                vals_v.at[pl.ds(j * BPW, 2 * BPW)],
                sem,
            )
        )
    for c in copies:
        c.wait()
    for c in range(BPW // L):
        s = vals_v[pl.ds(c * L, L)]
        for j in range(1, N_FIELDS):
            s = s + vals_v[pl.ds(j * BPW + c * L, L)]
        acc_v[pl.ds(c * L, L)] = s
    pltpu.sync_copy(acc_v, out_hbm.at[pl.ds(base, BPW)])


def kernel(X, table):
    xt = X.T.reshape(N_FIELDS, B)
    out = _lookup_sum(xt, table.reshape(1, -1))
    return out.reshape(B, 1)


# speedup vs baseline: 1.0549x; 1.0322x over previous
import functools

import jax
import jax.numpy as jnp
from jax import lax
from jax.experimental import pallas as pl
from jax.experimental.pallas import tpu as pltpu
from jax.experimental.pallas import tpu_sc as plsc

B = 16384
N_FIELDS = 26
NC = 2
NS = 16
NW = NC * NS
BPW = B // NW
L = 16


@functools.partial(
    pl.kernel,
    out_type=jax.ShapeDtypeStruct((B,), jnp.float32),
    mesh=plsc.VectorSubcoreMesh(core_axis_name="c", subcore_axis_name="s"),
    scratch_types=[
        pltpu.VMEM((N_FIELDS * BPW,), jnp.int32),
        pltpu.VMEM((N_FIELDS * BPW,), jnp.float32),
        pltpu.VMEM((BPW,), jnp.float32),
        pltpu.SemaphoreType.DMA,
    ],
)
def _lookup_sum(xt_hbm, tbl_hbm, out_hbm, idx_v, vals_v, acc_v, sem):
    wid = lax.axis_index("s") * NC + lax.axis_index("c")
    base = wid * BPW
    idx_copies = [
        pltpu.async_copy(
            xt_hbm.at[j, pl.ds(base, BPW)], idx_v.at[pl.ds(j * BPW, BPW)], sem
        )
        for j in range(N_FIELDS)
    ]
    for c in idx_copies:
        c.wait()
    copies = []
    for j in range(0, N_FIELDS, 2):
        copies.append(
            pltpu.async_copy(
                tbl_hbm.at[0].at[idx_v.at[pl.ds(j * BPW, 2 * BPW)]],
                vals_v.at[pl.ds(j * BPW, 2 * BPW)],
                sem,
            )
        )
    for p, cp in enumerate(copies):
        cp.wait()
        j = 2 * p
        for c in range(BPW // L):
            s = vals_v[pl.ds(j * BPW + c * L, L)]
            s = s + vals_v[pl.ds((j + 1) * BPW + c * L, L)]
            if p == 0:
                acc_v[pl.ds(c * L, L)] = s
            else:
                acc_v[pl.ds(c * L, L)] = acc_v[pl.ds(c * L, L)] + s
    pltpu.sync_copy(acc_v, out_hbm.at[pl.ds(base, BPW)])


def kernel(X, table):
    xt = X.T.reshape(N_FIELDS, B)
    out = _lookup_sum(xt, table.reshape(1, -1))
    return out.reshape(B, 1)
